# baseline (speedup 1.0000x reference)
import jax
import jax.numpy as jnp
from jax import lax
from jax.experimental import pallas as pl
from jax.experimental.pallas import tpu as pltpu

B, N, M, D = 8, 2048, 2048, 64
K = 128


def _one_batch(xv, yv):
    f32 = jnp.float32
    bf16 = jnp.bfloat16
    x2 = jnp.sum(xv * xv, axis=1, keepdims=True)
    y2 = jnp.sum(yv * yv, axis=1, keepdims=True)
    x2_hi = x2.astype(bf16)
    x2_lo = (x2 - x2_hi.astype(f32)).astype(bf16)
    y2_hi = y2.astype(bf16)
    y2_lo = (y2 - y2_hi.astype(f32)).astype(bf16)
    one_col = jnp.ones((N, 2), bf16)
    zpad = jnp.zeros((N, K - D - 4), bf16)
    xa = jnp.concatenate(
        [(-2.0 * xv).astype(bf16), x2_hi, x2_lo, one_col, zpad], axis=1)
    ya = jnp.concatenate(
        [yv.astype(bf16), one_col, y2_hi, y2_lo, zpad], axis=1)

    d = lax.dot_general(xa, ya, (((1,), (1,)), ((), ())),
                        preferred_element_type=f32)

    pm = d[:, 0:128]
    for k in range(1, M // 128):
        pm = jnp.minimum(pm, d[:, k * 128:(k + 1) * 128])
    rm = jnp.min(pm, axis=1)

    cm = jnp.min(d, axis=0)

    return (jnp.sum(jnp.maximum(cm, 0.0)) * (1.0 / (M * B))
            + jnp.sum(jnp.maximum(rm, 0.0)) * (1.0 / (N * B)))


def _chamfer_kernel(x_ref, y_ref, acc_ref):
    s = pl.program_id(0)

    @pl.when(s == 0)
    def _():
        acc_ref[0, 0] = 0.0

    acc_ref[0, 0] += (
        _one_batch(x_ref[0], y_ref[0]) + _one_batch(x_ref[1], y_ref[1])
        + _one_batch(x_ref[2], y_ref[2]) + _one_batch(x_ref[3], y_ref[3]))


@jax.jit
def kernel(x, y):
    acc = pl.pallas_call(
        _chamfer_kernel,
        grid=(B // 4,),
        in_specs=[
            pl.BlockSpec((4, N, D), lambda s: (s, 0, 0)),
            pl.BlockSpec((4, M, D), lambda s: (s, 0, 0)),
        ],
        out_specs=pl.BlockSpec(
            (1, 1), lambda s: (0, 0), memory_space=pltpu.SMEM),
        out_shape=jax.ShapeDtypeStruct((1, 1), jnp.float32),
    )(x, y)
    return acc[0, 0]


# speedup vs baseline: 1.0491x; 1.0491x over previous
import jax
import jax.numpy as jnp
from jax import lax
from jax.experimental import pallas as pl
from jax.experimental.pallas import tpu as pltpu

B, N, M, D = 8, 2048, 2048, 64
K = 128


def _one_batch(xv, yv):
    f32 = jnp.float32
    bf16 = jnp.bfloat16
    x2 = jnp.sum(xv * xv, axis=1, keepdims=True)
    y2 = jnp.sum(yv * yv, axis=1, keepdims=True)
    x2_hi = x2.astype(bf16)
    x2_lo = (x2 - x2_hi.astype(f32)).astype(bf16)
    y2_hi = y2.astype(bf16)
    y2_lo = (y2 - y2_hi.astype(f32)).astype(bf16)
    one_col = jnp.ones((N, 2), bf16)
    zpad = jnp.zeros((N, K - D - 4), bf16)
    xa = jnp.concatenate(
        [(-2.0 * xv).astype(bf16), x2_hi, x2_lo, one_col, zpad], axis=1)
    ya = jnp.concatenate(
        [yv.astype(bf16), one_col, y2_hi, y2_lo, zpad], axis=1)

    d = lax.dot_general(xa, ya, (((1,), (1,)), ((), ())),
                        preferred_element_type=f32)

    pm = d[:, 0:128]
    for k in range(1, M // 128):
        pm = jnp.minimum(pm, d[:, k * 128:(k + 1) * 128])
    rm = jnp.min(pm, axis=1)

    cm = jnp.min(d, axis=0)

    return (jnp.sum(jnp.maximum(cm, 0.0)) * (1.0 / (M * B))
            + jnp.sum(jnp.maximum(rm, 0.0)) * (1.0 / (N * B)))


def _chamfer_kernel(x_ref, y_ref, acc_ref):
    s = pl.program_id(0)

    @pl.when(s == 0)
    def _():
        acc_ref[0, 0] = 0.0

    acc_ref[0, 0] += _one_batch(x_ref[0], y_ref[0]) + _one_batch(
        x_ref[1], y_ref[1])


@jax.jit
def kernel(x, y):
    acc = pl.pallas_call(
        _chamfer_kernel,
        grid=(B // 2,),
        in_specs=[
            pl.BlockSpec((2, N, D), lambda s: (s, 0, 0)),
            pl.BlockSpec((2, M, D), lambda s: (s, 0, 0)),
        ],
        out_specs=pl.BlockSpec(
            (1, 1), lambda s: (0, 0), memory_space=pltpu.SMEM),
        out_shape=jax.ShapeDtypeStruct((1, 1), jnp.float32),
    )(x, y)
    return acc[0, 0]
